# Initial kernel scaffold; baseline (speedup 1.0000x reference)
import jax
import jax.numpy as jnp
from jax.experimental import pallas as pl


def kernel(x, edge_index, edge_attr, u, params):
    raise NotImplementedError("write your pallas kernel here")


# speedup vs baseline: 1.2292x; 1.2292x over previous
import functools
import jax
import jax.numpy as jnp
from jax import lax
from jax.experimental import pallas as pl
from jax.experimental.pallas import tpu as pltpu

F32 = jnp.float32


def _ln(h, g, b):
    m = jnp.mean(h, axis=-1, keepdims=True)
    v = jnp.mean((h - m) ** 2, axis=-1, keepdims=True)
    return (h - m) * lax.rsqrt(v + 1e-5) * g + b


def _dot(a, b):
    return jnp.dot(a, b, preferred_element_type=jnp.float32)


def _edge_body(xr, xc, ea, u,
               wxr, wxc, wea, wu, b1, w2, b2, g2, bt2,
               mxc, me, mb1, m2, mb2, mg, mbt,
               e_out, m_out):
    h = _dot(xr[...], wxr[...]) + _dot(xc[...], wxc[...]) \
        + _dot(ea[...], wea[...]) + _dot(u[...], wu[...]) + b1[...]
    h = jnp.maximum(h, 0.0)
    e = _ln(_dot(h, w2[...]) + b2[...], g2[...], bt2[...])
    e_out[...] = e
    hm = jnp.maximum(_dot(xc[...], mxc[...]) + _dot(e, me[...]) + mb1[...], 0.0)
    m_out[...] = _ln(_dot(hm, m2[...]) + mb2[...], mg[...], mbt[...])


def _edge_call(xr, xc, ea, u, pe, pm, BE=2048):
    EP, H = xr.shape
    FE = ea.shape[1]
    F = xr.shape[1]
    grid = EP // BE
    row_spec = lambda w: pl.BlockSpec((BE, w), lambda i: (i, 0))
    full = lambda a: pl.BlockSpec(a.shape, lambda i: (0, 0))
    wxr, wxc, wea, wu = (pe["l1"]["W"][:F], pe["l1"]["W"][F:2 * F],
                         pe["l1"]["W"][2 * F:2 * F + FE], pe["l1"]["W"][2 * F + FE:])
    mxc, me = pm["l1"]["W"][:F], pm["l1"]["W"][F:]
    r2 = lambda a: a.reshape(1, -1)
    args = (xr, xc, ea, u,
            wxr, wxc, wea, wu, r2(pe["l1"]["b"]), pe["l2"]["W"], r2(pe["l2"]["b"]),
            r2(pe["ln_g"]), r2(pe["ln_b"]),
            mxc, me, r2(pm["l1"]["b"]), pm["l2"]["W"], r2(pm["l2"]["b"]),
            r2(pm["ln_g"]), r2(pm["ln_b"]))
    in_specs = [row_spec(F), row_spec(F), row_spec(FE), row_spec(u.shape[1])] + \
               [full(a) for a in args[4:]]
    out_shape = [jax.ShapeDtypeStruct((EP, H), F32)] * 2
    out_specs = [row_spec(H), row_spec(H)]
    return pl.pallas_call(
        _edge_body, grid=(grid,), in_specs=in_specs,
        out_specs=out_specs, out_shape=out_shape)(*args)


def _node_body(x, s, cnt, u,
               wx, wa, wu2, b1, w2, b2, g, bt,
               g1w1, g1b1, g1w2, g1b2, g1g, g1bt,
               x_out, u1_out, cs_out, *, BN, nreal):
    i = pl.program_id(0)
    agg = s[...] / jnp.maximum(cnt[...], 1.0)
    h = jnp.maximum(_dot(x[...], wx[...]) + _dot(agg, wa[...])
                    + _dot(u[...], wu2[...]) + b1[...], 0.0)
    xn = _ln(_dot(h, w2[...]) + b2[...], g[...], bt[...])
    x_out[...] = xn
    rowid = i * BN + lax.broadcasted_iota(jnp.int32, xn.shape, 0)
    xm = jnp.where(rowid < nreal, xn, 0.0)

    @pl.when(i == 0)
    def _():
        cs_out[...] = jnp.zeros_like(cs_out)

    cs_out[...] += jnp.sum(xm, axis=0, keepdims=True)
    h1 = jnp.maximum(_dot(u[...], g1w1[...]) + g1b1[...], 0.0)
    u1_out[...] = _ln(_dot(h1, g1w2[...]) + g1b2[...], g1g[...], g1bt[...])


def _node_call(nreal, x, s, cnt, u, pn, pg1, BN=2048):
    NP, H = x.shape
    F = x.shape[1]
    grid = NP // BN
    row_spec = lambda w: pl.BlockSpec((BN, w), lambda i: (i, 0))
    full = lambda a: pl.BlockSpec(a.shape, lambda i: (0, 0))
    r2 = lambda a: a.reshape(1, -1)
    wx, wa, wu2 = pn["l1"]["W"][:F], pn["l1"]["W"][F:F + H], pn["l1"]["W"][F + H:]
    args = (x, s, cnt, u,
            wx, wa, wu2, r2(pn["l1"]["b"]), pn["l2"]["W"], r2(pn["l2"]["b"]),
            r2(pn["ln_g"]), r2(pn["ln_b"]),
            pg1["l1"]["W"], r2(pg1["l1"]["b"]), pg1["l2"]["W"], r2(pg1["l2"]["b"]),
            r2(pg1["ln_g"]), r2(pg1["ln_b"]))
    in_specs = [row_spec(F), row_spec(H), row_spec(1), row_spec(u.shape[1])] + \
               [full(a) for a in args[4:]]
    out_shape = [jax.ShapeDtypeStruct((NP, H), F32),
                 jax.ShapeDtypeStruct((NP, H), F32),
                 jax.ShapeDtypeStruct((1, H), F32)]
    out_specs = [row_spec(H), row_spec(H), pl.BlockSpec((1, H), lambda i: (0, 0))]
    return pl.pallas_call(
        functools.partial(_node_body, BN=BN, nreal=nreal),
        grid=(grid,), in_specs=in_specs,
        out_specs=out_specs, out_shape=out_shape)(*args)


def _glob_body(u1, eagg, ns, wa, wc, wb, b1, w2, b2, g, bt, u_out):
    h = _dot(u1[...], wa[...]) + _dot(eagg[...], wc[...]) \
        + _dot(ns[...], wb[...]) + b1[...]
    h = jnp.maximum(h, 0.0)
    u_out[...] = _ln(_dot(h, w2[...]) + b2[...], g[...], bt[...])


def _glob_call(u1, eagg, ns, pg2, BN=2048):
    NP, H = u1.shape
    grid = NP // BN
    row_spec = pl.BlockSpec((BN, H), lambda i: (i, 0))
    full = lambda a: pl.BlockSpec(a.shape, lambda i: (0, 0))
    r2 = lambda a: a.reshape(1, -1)
    wa, wb, wc = pg2["l1"]["W"][:H], pg2["l1"]["W"][H:2 * H], pg2["l1"]["W"][2 * H:]
    args = (u1, eagg, ns, wa, wc, wb, r2(pg2["l1"]["b"]), pg2["l2"]["W"],
            r2(pg2["l2"]["b"]), r2(pg2["ln_g"]), r2(pg2["ln_b"]))
    in_specs = [row_spec, row_spec, pl.BlockSpec((1, H), lambda i: (0, 0))] + \
               [full(a) for a in args[3:]]
    return pl.pallas_call(
        _glob_body, grid=(grid,), in_specs=in_specs,
        out_specs=row_spec,
        out_shape=jax.ShapeDtypeStruct((NP, H), F32))(*args)


def kernel(x, edge_index, edge_attr, u, params):
    N, F = x.shape
    E = edge_index.shape[1]
    H = params[0]["edge"]["l2"]["W"].shape[1]
    BE = 2048
    EP = -(-E // BE) * BE
    NP = -(-N // BE) * BE

    row = edge_index[0].astype(jnp.int32)
    col = edge_index[1].astype(jnp.int32)

    xp = jnp.zeros((NP, F), F32).at[:N].set(x)
    up = jnp.zeros((NP, u.shape[1]), F32).at[:N].set(u)
    eap = jnp.zeros((EP, edge_attr.shape[1]), F32).at[:E].set(edge_attr)
    rowp = jnp.zeros((EP,), jnp.int32).at[:E].set(row)
    colp = jnp.zeros((EP,), jnp.int32).at[:E].set(col)

    for p in params:
        xr = jnp.take(xp, rowp, axis=0)
        xc = jnp.take(xp, colp, axis=0)
        e_new, m = _edge_call(xr, xc, eap, up, p["edge"], p["node1"])
        sums = jax.ops.segment_sum(m[:E], row, num_segments=NP)
        cnt = jax.ops.segment_sum(jnp.ones((E, 1), F32), row, num_segments=NP)
        eagg = jax.ops.segment_sum(e_new[:E], row, num_segments=NP)
        xp, u1, ns = _node_call(N, xp, sums, cnt, up, p["node2"], p["glob1"])
        up = _glob_call(u1, eagg, ns, p["glob2"])
        eap = e_new

    return (xp[:N], eap[:E], up[:N])
